# Initial kernel scaffold; baseline (speedup 1.0000x reference)
import jax
import jax.numpy as jnp
from jax.experimental import pallas as pl


def kernel(x, edge_index, batch, Wq1, bq1, Wk1, bk1, Wv1, bv1, Ws1, bs1, Wq2, bq2, Wk2, bk2, Wv2, bv2, Ws2, bs2, W1, b1):
    raise NotImplementedError("write your pallas kernel here")


# speedup vs baseline: 22.3464x; 22.3464x over previous
import functools
import math

import jax
import jax.numpy as jnp
from jax import lax
from jax.experimental import pallas as pl
from jax.experimental.pallas import tpu as pltpu
from jax.experimental.pallas import tpu_sc as plsc

N = 100000
NP = 100352
E = 3200000
G = 1024
G2 = 1040
C = 1024
KSUB = 8
NCHUNK = E // C
NC = 2
NS = 16
NW = NC * NS
ROWS_T = NP // NS
BN = 6272

_mesh = plsc.VectorSubcoreMesh(core_axis_name="c", subcore_axis_name="s")


def _proj1_body(x_ref, w_ref, b_ref, q_ref, k_ref, v_ref, s_ref):
    h = jnp.dot(x_ref[...], w_ref[...], preferred_element_type=jnp.float32)
    h = h + b_ref[...]
    q_ref[...] = h[:, 0:16]
    k_ref[...] = h[:, 16:32]
    v_ref[...] = h[:, 32:48]
    s_ref[...] = h[:, 48:64]


def _proj1(x_p, wcat, bcat):
    o16 = jax.ShapeDtypeStruct((NP, 16), jnp.float32)
    return pl.pallas_call(
        _proj1_body,
        grid=(NP // BN,),
        in_specs=[
            pl.BlockSpec((BN, 11), lambda i: (i, 0)),
            pl.BlockSpec((11, 64), lambda i: (0, 0)),
            pl.BlockSpec((1, 64), lambda i: (0, 0)),
        ],
        out_specs=[pl.BlockSpec((BN, 16), lambda i: (i, 0))] * 4,
        out_shape=[o16, o16, o16, o16],
    )(x_p, wcat, bcat)


def _comb2_body(num_ref, den_ref, s_ref, w_ref, b_ref,
                q_ref, k_ref, va_ref, vb_ref, sa_ref, sb_ref):
    num = num_ref[0] + num_ref[1]
    den = den_ref[0] + den_ref[1] + 1e-16
    h = jnp.maximum(num / den[:, None] + s_ref[...], 0.0)
    p = jnp.dot(h, w_ref[...], preferred_element_type=jnp.float32)
    p = p + b_ref[...]
    q_ref[...] = p[:, 0:32]
    k_ref[...] = p[:, 32:64]
    va_ref[...] = p[:, 64:80]
    vb_ref[...] = p[:, 80:96]
    sa_ref[...] = p[:, 96:112]
    sb_ref[...] = p[:, 112:128]


def _comb_proj2(num1, den1, s1, wcat2, bcat2):
    o16 = jax.ShapeDtypeStruct((NP, 16), jnp.float32)
    o32 = jax.ShapeDtypeStruct((NP, 32), jnp.float32)
    return pl.pallas_call(
        _comb2_body,
        grid=(NP // BN,),
        in_specs=[
            pl.BlockSpec((2, BN, 16), lambda i: (0, i, 0)),
            pl.BlockSpec((2, BN), lambda i: (0, i)),
            pl.BlockSpec((BN, 16), lambda i: (i, 0)),
            pl.BlockSpec((16, 128), lambda i: (0, 0)),
            pl.BlockSpec((1, 128), lambda i: (0, 0)),
        ],
        out_specs=[
            pl.BlockSpec((BN, 32), lambda i: (i, 0)),
            pl.BlockSpec((BN, 32), lambda i: (i, 0)),
            pl.BlockSpec((BN, 16), lambda i: (i, 0)),
            pl.BlockSpec((BN, 16), lambda i: (i, 0)),
            pl.BlockSpec((BN, 16), lambda i: (i, 0)),
            pl.BlockSpec((BN, 16), lambda i: (i, 0)),
        ],
        out_shape=[o32, o32, o16, o16, o16, o16],
    )(num1, den1, s1, wcat2, bcat2)


def _gather_rows(tab_hbm, idx_ref, rows_ref, sem):
    cps = [pltpu.async_copy(tab_hbm.at[idx_ref.at[j]],
                            rows_ref.at[pl.ds(j * 128, 128)], sem)
           for j in range(KSUB)]
    for cp in cps:
        cp.wait()


def _make_alpha(d):
    inv = 1.0 / math.sqrt(float(d))

    def body(src_hbm, dst_hbm, q_hbm, k_hbm, alpha_hbm, wmax_hbm,
             srcv, dstv, qv, kv, av, mv, sem1, sem2):
        cid = lax.axis_index("c")
        sid = lax.axis_index("s")
        w = sid * NC + cid
        it = lax.iota(jnp.int32, 16)

        def chunk(t, m):
            c = t * NW + w

            def do(mm):
                pltpu.sync_copy(src_hbm.at[c], srcv)
                pltpu.sync_copy(dst_hbm.at[c], dstv)
                _gather_rows(q_hbm, dstv, qv, sem1)
                _gather_rows(k_hbm, srcv, kv, sem2)

                def grp(g, mg):
                    e0 = g * 16
                    rows = it + e0
                    acc = jnp.zeros((16,), jnp.float32)
                    for j in range(d):
                        colj = jnp.full((16,), j, jnp.int32)
                        qc = plsc.load_gather(qv, [rows, colj])
                        kc = plsc.load_gather(kv, [rows, colj])
                        acc = acc + qc * kc
                    a = acc * inv
                    av[pl.ds(e0, 16)] = a
                    return jnp.maximum(mg, jnp.max(a))

                mm = lax.fori_loop(0, C // 16, grp, mm)
                pltpu.sync_copy(av, alpha_hbm.at[c])
                return mm

            return lax.cond(c < NCHUNK, do, lambda mm: mm, m)

        m0 = jnp.float32(-3.0e38)
        m = lax.fori_loop(0, (NCHUNK + NW - 1) // NW, chunk, m0)
        mv[...] = jnp.full((16,), m)
        pltpu.sync_copy(mv, wmax_hbm.at[w])

    return functools.partial(
        pl.kernel, body,
        out_type=[jax.ShapeDtypeStruct((NCHUNK, C), jnp.float32),
                  jax.ShapeDtypeStruct((NW, 16), jnp.float32)],
        mesh=_mesh,
        compiler_params=pltpu.CompilerParams(needs_layout_passes=False, use_tc_tiling_on_sc=False),
        scratch_types=[
            pltpu.VMEM((KSUB, 128), jnp.int32),
            pltpu.VMEM((KSUB, 128), jnp.int32),
            pltpu.VMEM((C, d), jnp.float32),
            pltpu.VMEM((C, d), jnp.float32),
            pltpu.VMEM((C,), jnp.float32),
            pltpu.VMEM((16,), jnp.float32),
            pltpu.SemaphoreType.DMA,
            pltpu.SemaphoreType.DMA,
        ])()


_alpha16 = _make_alpha(16)
_alpha32 = _make_alpha(32)


def _global_max(wv):
    acc = wv[0]
    for i in range(1, NW):
        acc = jnp.maximum(acc, wv[i])
    return jnp.max(acc)


def _zero_accum(sid, z2_hbm, z1_hbm, num_sh, den_sh):
    for r in range(2):
        pltpu.sync_copy(z2_hbm, num_sh.at[pl.ds(sid * ROWS_T + r * 3136, 3136)])
    pltpu.sync_copy(z1_hbm, den_sh.at[pl.ds(sid * ROWS_T, ROWS_T)])


def _dump_accum(sid, cid, num_sh, den_sh, num_hbm, den_hbm,
                den_2d, do_den):
    sl = pl.ds(sid * ROWS_T, ROWS_T)
    pltpu.sync_copy(num_sh.at[sl], num_hbm.at[cid, sl])
    if do_den:
        if den_2d:
            pltpu.sync_copy(den_sh.at[sl], den_hbm.at[cid, sl])
        else:
            pltpu.sync_copy(den_sh.at[sl], den_hbm.at[sl])


def _edge_accum_chunk(c, gmax, src_hbm, dst_hbm, alpha_hbm, vtab_hbm,
                      srcv, dstv, vrows, av, exf, sem, num_sh, den_sh, do_den):
    it = lax.iota(jnp.int32, 16)
    pltpu.sync_copy(src_hbm.at[c], srcv)
    pltpu.sync_copy(dst_hbm.at[c], dstv)
    pltpu.sync_copy(alpha_hbm.at[c], av)
    _gather_rows(vtab_hbm, srcv, vrows, sem)

    def grp(g, _):
        e0 = g * 16
        rows = it + e0
        a = av[pl.ds(e0, 16)]
        ex = jnp.exp(a - gmax)
        exf[pl.ds(e0, 16)] = ex
        for j in range(16):
            colj = jnp.full((16,), j, jnp.int32)
            v = plsc.load_gather(vrows, [rows, colj])
            plsc.store_scatter(vrows, [rows, colj], v * ex)
        return 0

    lax.fori_loop(0, C // 16, grp, 0)
    for j in range(KSUB):
        pltpu.sync_copy(vrows.at[pl.ds(j * 128, 128)],
                        num_sh.at[dstv.at[j]], add=True)
        if do_den:
            pltpu.sync_copy(exf.at[pl.ds(j * 128, 128)],
                            den_sh.at[dstv.at[j]], add=True)


def _scatter1_body(src_hbm, dst_hbm, alpha_hbm, v_hbm, wmax_hbm, z2_hbm, z1_hbm,
                   num_hbm, den_hbm,
                   srcv, dstv, vrows, av, exf, wv, sem, semw,
                   num_sh, den_sh):
    cid = lax.axis_index("c")
    sid = lax.axis_index("s")
    w = sid * NC + cid
    _zero_accum(sid, z2_hbm, z1_hbm, num_sh, den_sh)
    pltpu.sync_copy(wmax_hbm, wv)
    gmax = _global_max(wv)
    plsc.subcore_barrier()

    def chunk(t, _):
        c = t * NW + w

        @pl.when(c < NCHUNK)
        def _():
            _edge_accum_chunk(c, gmax, src_hbm, dst_hbm, alpha_hbm, v_hbm,
                              srcv, dstv, vrows, av, exf, sem,
                              num_sh, den_sh, True)
        return 0

    lax.fori_loop(0, (NCHUNK + NW - 1) // NW, chunk, 0)
    plsc.subcore_barrier()
    _dump_accum(sid, cid, num_sh, den_sh, num_hbm, den_hbm, True, True)


def _scatter1(srcR, dstR, alphaR, v1, wmax, z2, z1):
    return pl.kernel(
        _scatter1_body,
        out_type=[jax.ShapeDtypeStruct((NC, NP, 16), jnp.float32),
                  jax.ShapeDtypeStruct((NC, NP), jnp.float32)],
        mesh=_mesh,
        compiler_params=pltpu.CompilerParams(needs_layout_passes=False, use_tc_tiling_on_sc=False),
        scratch_types=[
            pltpu.VMEM((KSUB, 128), jnp.int32),
            pltpu.VMEM((KSUB, 128), jnp.int32),
            pltpu.VMEM((C, 16), jnp.float32),
            pltpu.VMEM((C,), jnp.float32),
            pltpu.VMEM((C,), jnp.float32),
            pltpu.VMEM((NW, 16), jnp.float32),
            pltpu.SemaphoreType.DMA,
            pltpu.SemaphoreType.DMA,
            pltpu.VMEM_SHARED((NP, 16), jnp.float32),
            pltpu.VMEM_SHARED((NP,), jnp.float32),
        ])(srcR, dstR, alphaR, v1, wmax, z2, z1)


def _scatter2_body(src_hbm, dst_hbm, alpha_hbm, va_hbm, vb_hbm, wmax_hbm,
                   z2_hbm, z1_hbm,
                   num_hbm, den_hbm,
                   srcv, dstv, vrows, av, exf, wv, sem, semw,
                   num_sh, den_sh):
    cid = lax.axis_index("c")
    sid = lax.axis_index("s")
    _zero_accum(sid, z2_hbm, z1_hbm, num_sh, den_sh)
    pltpu.sync_copy(wmax_hbm, wv)
    gmax = _global_max(wv)
    plsc.subcore_barrier()

    def sweep(vtab_hbm, do_den):
        def chunk(t, _):
            c = t * NS + sid

            @pl.when(c < NCHUNK)
            def _():
                _edge_accum_chunk(c, gmax, src_hbm, dst_hbm, alpha_hbm,
                                  vtab_hbm, srcv, dstv, vrows, av, exf, sem,
                                  num_sh, den_sh, do_den)
            return 0

        lax.fori_loop(0, (NCHUNK + NS - 1) // NS, chunk, 0)

    @pl.when(cid == 0)
    def _():
        sweep(va_hbm, True)

    @pl.when(cid == 1)
    def _():
        sweep(vb_hbm, False)

    plsc.subcore_barrier()
    _dump_accum2 = functools.partial(_dump_accum, sid, cid,
                                     num_sh, den_sh, num_hbm, den_hbm, False)

    @pl.when(cid == 0)
    def _():
        _dump_accum2(True)

    @pl.when(cid == 1)
    def _():
        _dump_accum2(False)


def _scatter2(srcR, dstR, alphaR, v2a, v2b, wmax, z2, z1):
    return pl.kernel(
        _scatter2_body,
        out_type=[jax.ShapeDtypeStruct((NC, NP, 16), jnp.float32),
                  jax.ShapeDtypeStruct((NP,), jnp.float32)],
        mesh=_mesh,
        compiler_params=pltpu.CompilerParams(needs_layout_passes=False, use_tc_tiling_on_sc=False),
        scratch_types=[
            pltpu.VMEM((KSUB, 128), jnp.int32),
            pltpu.VMEM((KSUB, 128), jnp.int32),
            pltpu.VMEM((C, 16), jnp.float32),
            pltpu.VMEM((C,), jnp.float32),
            pltpu.VMEM((C,), jnp.float32),
            pltpu.VMEM((NW, 16), jnp.float32),
            pltpu.SemaphoreType.DMA,
            pltpu.SemaphoreType.DMA,
            pltpu.VMEM_SHARED((NP, 16), jnp.float32),
            pltpu.VMEM_SHARED((NP,), jnp.float32),
        ])(srcR, dstR, alphaR, v2a, v2b, wmax, z2, z1)


_PCH = 896


def _pool_body(num_hbm, den_hbm, sa_hbm, sb_hbm, batch_hbm, w1a_hbm, w1b_hbm,
               b1_hbm, iota_hbm, z1_hbm,
               out_hbm,
               na, nb, sa, sb, denv, bv, zacc, cacc, w1av, w1bv, b1v, iotav,
               zbufd, sem,
               zsum_sh, cnt_sh):
    cid = lax.axis_index("c")
    sid = lax.axis_index("s")

    @pl.when(cid == 0)
    def _():
        @pl.when(sid == 0)
        def _():
            pltpu.sync_copy(z1_hbm.at[pl.ds(0, G2)], zbufd)
            pltpu.sync_copy(zbufd, zsum_sh)
            pltpu.sync_copy(zbufd, cnt_sh)
        pltpu.sync_copy(w1a_hbm, w1av)
        pltpu.sync_copy(w1b_hbm, w1bv)
        pltpu.sync_copy(b1_hbm, b1v)
        pltpu.sync_copy(iota_hbm, iotav)
        z16 = jnp.zeros((16,), jnp.float32)

        def zc(g, _):
            zacc[pl.ds(g * 16, 16)] = z16
            cacc[pl.ds(g * 16, 16)] = z16
            return 0

        lax.fori_loop(0, G2 // 16, zc, 0)
        plsc.subcore_barrier()
        w1a = w1av[...]
        w1b = w1bv[...]
        ones16 = jnp.ones((16,), jnp.float32)
        it = lax.iota(jnp.int32, 16)

        for k in range(ROWS_T // _PCH):
            base = sid * ROWS_T + k * _PCH
            pltpu.sync_copy(num_hbm.at[0, pl.ds(base, _PCH)], na)
            pltpu.sync_copy(num_hbm.at[1, pl.ds(base, _PCH)], nb)
            pltpu.sync_copy(den_hbm.at[pl.ds(base, _PCH)], denv)
            pltpu.sync_copy(sa_hbm.at[pl.ds(base, _PCH)], sa)
            pltpu.sync_copy(sb_hbm.at[pl.ds(base, _PCH)], sb)
            pltpu.sync_copy(batch_hbm.at[pl.ds(base, _PCH)], bv)

            def grp(g, _):
                e0 = g * 16
                rows = it + e0
                dd = denv[pl.ds(e0, 16)] + 1e-16
                b16 = bv[pl.ds(e0, 16)]
                zs = jnp.zeros((16,), jnp.float32)
                for j in range(16):
                    colj = jnp.full((16,), j, jnp.int32)
                    va = plsc.load_gather(na, [rows, colj])
                    ra = plsc.load_gather(sa, [rows, colj])
                    vb = plsc.load_gather(nb, [rows, colj])
                    rb = plsc.load_gather(sb, [rows, colj])
                    ha = jnp.maximum(va / dd + ra, 0.0)
                    hb = jnp.maximum(vb / dd + rb, 0.0)
                    zs = zs + ha * w1a[j] + hb * w1b[j]
                plsc.addupdate_scatter(zacc, [b16], zs)
                plsc.addupdate_scatter(cacc, [b16], ones16)
                return 0

            lax.fori_loop(0, _PCH // 16, grp, 0)

        for j in range(8):
            pltpu.sync_copy(zacc.at[pl.ds(j * 128, 128)],
                            zsum_sh.at[iotav.at[j]], add=True)
            pltpu.sync_copy(cacc.at[pl.ds(j * 128, 128)],
                            cnt_sh.at[iotav.at[j]], add=True)
        plsc.subcore_barrier()

        @pl.when(sid == 0)
        def _():
            pltpu.sync_copy(zsum_sh.at[pl.ds(0, G)], zacc.at[pl.ds(0, G)])
            pltpu.sync_copy(cnt_sh.at[pl.ds(0, G)], cacc.at[pl.ds(0, G)])
            b1 = b1v[...]

            def fin(g, _):
                zs = zacc[pl.ds(g * 16, 16)]
                cn = jnp.maximum(cacc[pl.ds(g * 16, 16)], 1.0)
                zbufd[pl.ds(g * 16, 16)] = jnp.maximum(zs / cn + b1, 0.0)
                return 0

            lax.fori_loop(0, G // 16, fin, 0)
            pltpu.sync_copy(zbufd.at[pl.ds(0, G)], out_hbm)


def _pool(num2, den2, s2a, s2b, batch_p, w1a, w1b, b1v, iotaG, z1):
    return pl.kernel(
        _pool_body,
        out_type=jax.ShapeDtypeStruct((G,), jnp.float32),
        mesh=_mesh,
        compiler_params=pltpu.CompilerParams(needs_layout_passes=False, use_tc_tiling_on_sc=False),
        scratch_types=[
            pltpu.VMEM((_PCH, 16), jnp.float32),
            pltpu.VMEM((_PCH, 16), jnp.float32),
            pltpu.VMEM((_PCH, 16), jnp.float32),
            pltpu.VMEM((_PCH, 16), jnp.float32),
            pltpu.VMEM((_PCH,), jnp.float32),
            pltpu.VMEM((_PCH,), jnp.int32),
            pltpu.VMEM((G2,), jnp.float32),
            pltpu.VMEM((G2,), jnp.float32),
            pltpu.VMEM((16,), jnp.float32),
            pltpu.VMEM((16,), jnp.float32),
            pltpu.VMEM((16,), jnp.float32),
            pltpu.VMEM((8, 128), jnp.int32),
            pltpu.VMEM((G2,), jnp.float32),
            pltpu.SemaphoreType.DMA,
            pltpu.VMEM_SHARED((G2,), jnp.float32),
            pltpu.VMEM_SHARED((G2,), jnp.float32),
        ])(num2, den2, s2a, s2b, batch_p, w1a, w1b, b1v, iotaG, z1)


def kernel(x, edge_index, batch, Wq1, bq1, Wk1, bk1, Wv1, bv1, Ws1, bs1,
           Wq2, bq2, Wk2, bk2, Wv2, bv2, Ws2, bs2, W1, b1):
    src = edge_index[0]
    dst = edge_index[1]
    srcR = src.reshape(NCHUNK, KSUB, 128).astype(jnp.int32)
    dstR = dst.reshape(NCHUNK, KSUB, 128).astype(jnp.int32)
    x_p = jnp.pad(x, ((0, NP - N), (0, 0)))
    batch_p = jnp.concatenate(
        [batch.astype(jnp.int32), jnp.full((NP - N,), G, jnp.int32)])

    wcat1 = jnp.concatenate([Wq1, Wk1, Wv1, Ws1], axis=1)
    bcat1 = jnp.concatenate([bq1, bk1, bv1, bs1]).reshape(1, 64)
    wcat2 = jnp.concatenate([Wq2, Wk2, Wv2, Ws2], axis=1)
    bcat2 = jnp.concatenate([bq2, bk2, bv2, bs2]).reshape(1, 128)

    z2 = jnp.zeros((3136, 16), jnp.float32)
    z1 = jnp.zeros((ROWS_T,), jnp.float32)
    w1a = W1[0:16, 0]
    w1b = W1[16:32, 0]
    b1v = jnp.full((16,), b1[0], jnp.float32)
    iotaG = jnp.arange(G, dtype=jnp.int32).reshape(8, 128)

    q1, k1, v1, s1 = _proj1(x_p, wcat1, bcat1)
    alpha1, wmax1 = _alpha16(srcR, dstR, q1, k1)
    num1, den1 = _scatter1(srcR, dstR, alpha1, v1, wmax1, z2, z1)
    q2, k2, v2a, v2b, s2a, s2b = _comb_proj2(num1, den1, s1, wcat2, bcat2)
    alpha2, wmax2 = _alpha32(srcR, dstR, q2, k2)
    num2, den2 = _scatter2(srcR, dstR, alpha2, v2a, v2b, wmax2, z2, z1)
    out = _pool(num2, den2, s2a, s2b, batch_p, w1a, w1b, b1v, iotaG, z1)
    return out.reshape(G, 1)
